# baseline (speedup 1.0000x reference)
import functools

import jax
import jax.numpy as jnp
import numpy as np
from jax import lax
from jax.experimental import pallas as pl
from jax.experimental.pallas import tpu as pltpu
from jax.experimental.pallas import tpu_sc as plsc

N_NODES = 10000
D = 256
DI = D // 2
E = 160000
NS = 16
NC = 2
HALF_N = N_NODES // NC
E_PAD = 163840
EDGES_PER_SUB = E_PAD // NS
SCAN_ROWS = 8
SCAN_CHUNK = SCAN_ROWS * 128
N_CHUNKS = EDGES_PER_SUB // SCAN_CHUNK
CBUF = SCAN_CHUNK + 144
BLK = 64
ACC_NODES = HALF_N + 24
ACC_ROWS = 2 * ACC_NODES
INIT_ROWS = 320
INIT_ROWS_LAST = HALF_N - (NS - 1) * INIT_ROWS
ROW_BLK = 2000

_PERM = np.empty((D,), dtype=np.int32)
for _i in range(DI):
    _PERM[2 * _i] = _i
    _PERM[2 * _i + 1] = DI + _i


def _dense_body(x_ref, wc_ref, wl_ref, b_ref, h_ref, base_ref):
    xb = x_ref[...]
    dn = (((1,), (1,)), ((), ()))
    h = lax.dot_general(xb, wc_ref[...], dn, preferred_element_type=jnp.float32)
    base = lax.dot_general(xb, wl_ref[...], dn, preferred_element_type=jnp.float32)
    h_ref[...] = h.astype(jnp.bfloat16)
    base_ref[...] = base + b_ref[...]


def _dense(x, W_conv_perm, W_lin, b_lin):
    return pl.pallas_call(
        _dense_body,
        grid=(N_NODES // ROW_BLK,),
        in_specs=[
            pl.BlockSpec((ROW_BLK, D), lambda i: (i, 0)),
            pl.BlockSpec((D, D), lambda i: (0, 0)),
            pl.BlockSpec((D, D), lambda i: (0, 0)),
            pl.BlockSpec((1, D), lambda i: (0, 0)),
        ],
        out_specs=[
            pl.BlockSpec((ROW_BLK, D), lambda i: (i, 0)),
            pl.BlockSpec((ROW_BLK, D), lambda i: (i, 0)),
        ],
        out_shape=[
            jax.ShapeDtypeStruct((N_NODES, D), jnp.bfloat16),
            jax.ShapeDtypeStruct((N_NODES, D), jnp.float32),
        ],
    )(x, W_conv_perm, W_lin, b_lin.reshape(1, D))


_sc_mesh = plsc.VectorSubcoreMesh(core_axis_name="c", subcore_axis_name="s")


@functools.partial(
    pl.kernel,
    out_type=jax.ShapeDtypeStruct((2 * N_NODES, DI), jnp.float32),
    mesh=_sc_mesh,
    scratch_types=[
        pltpu.VMEM((SCAN_ROWS, 128), jnp.int32),
        pltpu.VMEM((SCAN_ROWS, 128), jnp.int32),
        pltpu.VMEM((CBUF,), jnp.int32),
        pltpu.VMEM((CBUF,), jnp.int32),
        pltpu.VMEM((8, 128), jnp.int32),
        pltpu.VMEM((BLK, DI), jnp.int32),
        pltpu.VMEM((2 * BLK, DI), jnp.float32),
        pltpu.VMEM_SHARED((ACC_ROWS, DI), jnp.float32),
        pltpu.SemaphoreType.DMA,
    ],
)
def _sc_agg(h_hbm, base_hbm, src_hbm, dst_hbm, out_hbm,
            src_v, dst_v, src_c, dst_c, idx_blk, gb, fb, acc_sh, gsem):
    c = lax.axis_index("c")
    s = lax.axis_index("s")
    base_row = c * HALF_N
    r0 = s * INIT_ROWS

    @pl.when(s < NS - 1)
    def _():
        pltpu.sync_copy(base_hbm.at[pl.ds(2 * (base_row + r0), 2 * INIT_ROWS)],
                        acc_sh.at[pl.ds(2 * r0, 2 * INIT_ROWS)])

    @pl.when(s == NS - 1)
    def _():
        pltpu.sync_copy(
            base_hbm.at[pl.ds(2 * (base_row + (NS - 1) * INIT_ROWS),
                              2 * INIT_ROWS_LAST)],
            acc_sh.at[pl.ds(2 * (NS - 1) * INIT_ROWS, 2 * INIT_ROWS_LAST)])

    plsc.subcore_barrier()

    idx_row0 = s * (EDGES_PER_SUB // 128)
    zeros16 = jnp.zeros((16,), jnp.int32)
    ones16 = jnp.full((16,), 1, jnp.int32)
    dump16 = jnp.full((16,), HALF_N, jnp.int32)
    himask = jnp.full((16,), -65536, jnp.int32)
    lane = lax.iota(jnp.int32, 16)
    lane_half = lane >> ones16
    lane_par = lane & ones16
    fifteen16 = jnp.full((16,), 15, jnp.int32)

    def take(v, i):
        return v.at[i].get(mode="promise_in_bounds")

    def chunk_body(k, carry):
        r = idx_row0 + k * SCAN_ROWS
        pltpu.sync_copy(src_hbm.at[pl.ds(r, SCAN_ROWS)], src_v)
        pltpu.sync_copy(dst_hbm.at[pl.ds(r, SCAN_ROWS)], dst_v)

        def fill(i, carry2):
            src_c[pl.ds(16 * i, 16)] = zeros16
            dst_c[pl.ds(16 * i, 16)] = dump16
            return carry2

        lax.fori_loop(0, CBUF // 16, fill, 0)

        def scan(i, cnt):
            row = i // 8
            gq = i % 8
            sv = src_v[row, pl.ds(16 * gq, 16)]
            dv = dst_v[row, pl.ds(16 * gq, 16)]
            local = dv - base_row
            m = (local >= 0) & (local < HALF_N)
            m_i32 = jnp.where(m, ones16, zeros16)
            ps = m_i32
            for sh in (1, 2, 4, 8):
                shv = jnp.full((16,), sh, jnp.int32)
                ok = lane >= shv
                contrib = take(ps, jnp.where(ok, lane - shv, zeros16))
                ps = ps + jnp.where(ok, contrib, zeros16)
            cg = take(ps, fifteen16)
            t = lane + ones16
            iv = zeros16
            for sh in (8, 4, 2, 1):
                shv = jnp.full((16,), sh, jnp.int32)
                probe = take(ps, iv + shv - ones16)
                iv = iv + jnp.where(probe < t, shv, zeros16)
            valid = t <= cg
            src_c[pl.ds(cnt, 16)] = jnp.where(valid, take(sv, iv), zeros16)
            dst_c[pl.ds(cnt, 16)] = jnp.where(valid, take(local, iv), dump16)
            return cnt + ps[15]

        cnt = lax.fori_loop(0, SCAN_ROWS * 8, scan, 0)
        nb = (cnt + BLK - 1) // BLK

        def block_body(b, carry3):
            for q in range(8):
                d16 = dst_c[pl.ds(BLK * b + 8 * q, 16)]
                pick = take(d16, lane_half)
                idx_blk[0, pl.ds(16 * q, 16)] = pick * 2 + lane_par
            pltpu.async_copy(
                h_hbm.at[src_c.at[pl.ds(BLK * b, BLK)]], gb, gsem).wait()

            def cvt(rr, carry4):
                for gq in range(DI // 16):
                    x = gb[rr, pl.ds(16 * gq, 16)]
                    lo = lax.bitcast_convert_type(x << 16, jnp.float32)
                    hi = lax.bitcast_convert_type(x & himask, jnp.float32)
                    fb[2 * rr, pl.ds(16 * gq, 16)] = lo
                    fb[2 * rr + 1, pl.ds(16 * gq, 16)] = hi
                return carry4

            lax.fori_loop(0, BLK, cvt, 0)
            pltpu.sync_copy(fb, acc_sh.at[idx_blk.at[0]], add=True)
            return carry3

        lax.fori_loop(0, nb, block_body, 0)
        return carry

    lax.fori_loop(0, N_CHUNKS, chunk_body, 0)
    plsc.subcore_barrier()

    @pl.when(s < NS - 1)
    def _():
        pltpu.sync_copy(acc_sh.at[pl.ds(2 * r0, 2 * INIT_ROWS)],
                        out_hbm.at[pl.ds(2 * (base_row + r0), 2 * INIT_ROWS)])

    @pl.when(s == NS - 1)
    def _():
        pltpu.sync_copy(
            acc_sh.at[pl.ds(2 * (NS - 1) * INIT_ROWS, 2 * INIT_ROWS_LAST)],
            out_hbm.at[pl.ds(2 * (base_row + (NS - 1) * INIT_ROWS),
                             2 * INIT_ROWS_LAST)])


def kernel(x, edge_index, W_conv, W_lin, b_lin):
    h_bf, base = _dense(x, W_conv[jnp.asarray(_PERM)], W_lin, b_lin)
    hbi = lax.bitcast_convert_type(
        h_bf.reshape(N_NODES, DI, 2), jnp.int32)
    src = edge_index[0].astype(jnp.int32)
    dst = edge_index[1].astype(jnp.int32)
    pad = E_PAD - E
    src_p = jnp.concatenate([src, jnp.zeros((pad,), jnp.int32)])
    dst_p = jnp.concatenate([dst, jnp.full((pad,), N_NODES, jnp.int32)])
    src2d = src_p.reshape(E_PAD // 128, 128)
    dst2d = dst_p.reshape(E_PAD // 128, 128)
    out = _sc_agg(hbi, base.reshape(2 * N_NODES, DI), src2d, dst2d)
    return out.reshape(N_NODES, D)


# speedup vs baseline: 1.4270x; 1.4270x over previous
import functools

import jax
import jax.numpy as jnp
import numpy as np
from jax import lax
from jax.experimental import pallas as pl
from jax.experimental.pallas import tpu as pltpu
from jax.experimental.pallas import tpu_sc as plsc

N_NODES = 10000
D = 256
DI = D // 2
E = 160000
NS = 16
NC = 2
QN = N_NODES // 4
QACC_NODES = QN + 8
ACC_ROWS = 2 * QACC_NODES
E_PAD = 163840
EDGES_PER_SUB = E_PAD // NS
SCAN_ROWS = 8
SCAN_CHUNK = SCAN_ROWS * 128
N_CHUNKS = EDGES_PER_SUB // SCAN_CHUNK
CBUF = EDGES_PER_SUB + 144
BLK = 64
INIT_ROWS = 320
INIT_ROWS_LAST = 2 * QN - (NS - 1) * INIT_ROWS
ROW_BLK = 2000

_PERM = np.empty((D,), dtype=np.int32)
for _i in range(DI):
    _PERM[2 * _i] = _i
    _PERM[2 * _i + 1] = DI + _i


def _dense_body(x_ref, wc_ref, wl_ref, b_ref, h_ref, base_ref):
    xb = x_ref[...]
    dn = (((1,), (1,)), ((), ()))
    h = lax.dot_general(xb, wc_ref[...], dn, preferred_element_type=jnp.float32)
    base = lax.dot_general(xb, wl_ref[...], dn, preferred_element_type=jnp.float32)
    h_ref[...] = h.astype(jnp.bfloat16)
    base_ref[...] = base + b_ref[...]


def _dense(x, W_conv_perm, W_lin, b_lin):
    return pl.pallas_call(
        _dense_body,
        grid=(N_NODES // ROW_BLK,),
        in_specs=[
            pl.BlockSpec((ROW_BLK, D), lambda i: (i, 0)),
            pl.BlockSpec((D, D), lambda i: (0, 0)),
            pl.BlockSpec((D, D), lambda i: (0, 0)),
            pl.BlockSpec((1, D), lambda i: (0, 0)),
        ],
        out_specs=[
            pl.BlockSpec((ROW_BLK, D), lambda i: (i, 0)),
            pl.BlockSpec((ROW_BLK, D), lambda i: (i, 0)),
        ],
        out_shape=[
            jax.ShapeDtypeStruct((N_NODES, D), jnp.bfloat16),
            jax.ShapeDtypeStruct((N_NODES, D), jnp.float32),
        ],
    )(x, W_conv_perm, W_lin, b_lin.reshape(1, D))


_sc_mesh = plsc.VectorSubcoreMesh(core_axis_name="c", subcore_axis_name="s")


@functools.partial(
    pl.kernel,
    out_type=jax.ShapeDtypeStruct((2 * N_NODES, DI), jnp.float32),
    mesh=_sc_mesh,
    scratch_types=[
        pltpu.VMEM((SCAN_ROWS, 128), jnp.int32),
        pltpu.VMEM((CBUF,), jnp.int32),
        pltpu.VMEM((CBUF,), jnp.int32),
        pltpu.VMEM((BLK, DI), jnp.int32),
        pltpu.VMEM((BLK, DI), jnp.int32),
        pltpu.VMEM((2 * BLK, DI), jnp.float32),
        pltpu.VMEM((2 * BLK, DI), jnp.float32),
        pltpu.VMEM_SHARED((ACC_ROWS, DI), jnp.float32),
        pltpu.SemaphoreType.DMA,
        pltpu.SemaphoreType.DMA,
        pltpu.SemaphoreType.DMA,
        pltpu.SemaphoreType.DMA,
    ],
)
def _sc_agg(h_hbm, base_hbm, edges_hbm, out_hbm,
            ev_v, src_c, dst_c, gb0, gb1, fb0, fb1, acc_sh,
            gsem0, gsem1, ssem0, ssem1):
    c = lax.axis_index("c")
    s = lax.axis_index("s")
    r0 = s * INIT_ROWS
    idx_row0 = s * (EDGES_PER_SUB // 128)

    zeros16 = jnp.zeros((16,), jnp.int32)
    ones16 = jnp.full((16,), 1, jnp.int32)
    two16 = jnp.full((16,), 2, jnp.int32)
    dump16 = jnp.full((16,), QN, jnp.int32)
    lomask = jnp.full((16,), 16383, jnp.int32)
    shift14 = jnp.full((16,), 14, jnp.int32)
    himask = jnp.full((16,), -65536, jnp.int32)
    lane = lax.iota(jnp.int32, 16)
    lane_half = lane >> ones16
    lane_par = lane & ones16
    fifteen16 = jnp.full((16,), 15, jnp.int32)

    gb = (gb0, gb1)
    fb = (fb0, fb1)
    gsems = (gsem0, gsem1)
    ssems = (ssem0, ssem1)

    def take(v, i):
        return v.at[i].get(mode="promise_in_bounds")

    for q in (0, 1):
        lo = c * (2 * QN) + q * QN

        @pl.when(s < NS - 1)
        def _():
            pltpu.sync_copy(base_hbm.at[pl.ds(2 * lo + r0, INIT_ROWS)],
                            acc_sh.at[pl.ds(r0, INIT_ROWS)])

        @pl.when(s == NS - 1)
        def _():
            pltpu.sync_copy(
                base_hbm.at[pl.ds(2 * lo + (NS - 1) * INIT_ROWS,
                                  INIT_ROWS_LAST)],
                acc_sh.at[pl.ds((NS - 1) * INIT_ROWS, INIT_ROWS_LAST)])

        plsc.subcore_barrier()

        def fill(i, carry2):
            src_c[pl.ds(16 * i, 16)] = zeros16
            dst_c[pl.ds(16 * i, 16)] = dump16
            return carry2

        lax.fori_loop(0, CBUF // 16, fill, 0)

        lo16 = ones16 * lo

        def chunk_scan(k, cnt0):
            pltpu.sync_copy(edges_hbm.at[pl.ds(idx_row0 + k * SCAN_ROWS,
                                               SCAN_ROWS)], ev_v)

            def scan(i, cnt):
                ev = ev_v[i // 8, pl.ds(16 * (i % 8), 16)]
                sv = ev & lomask
                local = (ev >> shift14) - lo16
                m = (local >= 0) & (local < dump16)
                m_i32 = jnp.where(m, ones16, zeros16)
                ps = m_i32
                for sh in (1, 2, 4, 8):
                    shv = jnp.full((16,), sh, jnp.int32)
                    ok = lane >= shv
                    contrib = take(ps, jnp.where(ok, lane - shv, zeros16))
                    ps = ps + jnp.where(ok, contrib, zeros16)
                cg = take(ps, fifteen16)
                t = lane + ones16
                iv = zeros16
                for sh in (8, 4, 2, 1):
                    shv = jnp.full((16,), sh, jnp.int32)
                    probe = take(ps, iv + shv - ones16)
                    iv = iv + jnp.where(probe < t, shv, zeros16)
                valid = t <= cg
                src_c[pl.ds(cnt, 16)] = jnp.where(valid, take(sv, iv), zeros16)
                dst_c[pl.ds(cnt, 16)] = jnp.where(valid, take(local, iv),
                                                  dump16)
                return cnt + ps[15]

            return lax.fori_loop(0, SCAN_ROWS * 8, scan, cnt0)

        cnt = lax.fori_loop(0, N_CHUNKS, chunk_scan, 0)
        nb = (cnt + BLK - 1) // BLK
        nbp = (nb + 1) // 2

        def start_gather(b, p):
            return pltpu.async_copy(
                h_hbm.at[src_c.at[pl.ds(BLK * b, BLK)]], gb[p], gsems[p])

        def wait_gather(b, p):
            pltpu.make_async_copy(
                h_hbm.at[src_c.at[pl.ds(BLK * b, BLK)]], gb[p],
                gsems[p]).wait()

        def start_scatter(p):
            return pltpu.async_copy(fb[p], acc_sh.at[ev_v.at[p]], ssems[p],
                                    add=True)

        def wait_scatter(p):
            pltpu.make_async_copy(fb[p], acc_sh.at[ev_v.at[p]],
                                  ssems[p]).wait()

        for p in (0, 1):
            @pl.when(p < nb)
            def _(p=p):
                start_gather(p, p)

        def pipe(j, carry3):
            for p in (0, 1):
                b = 2 * j + p
                cond = b < nb

                @pl.when(cond & (b >= 2))
                def _(p=p):
                    wait_scatter(p)

                @pl.when(cond)
                def _(b=b, p=p):
                    wait_gather(b, p)
                    for qq in range(8):
                        d16 = dst_c[pl.ds(BLK * b + 8 * qq, 16)]
                        pick = take(d16, lane_half)
                        ev_v[p, pl.ds(16 * qq, 16)] = pick * two16 + lane_par

                    def cvt(rr, carry4):
                        for gq in range(DI // 16):
                            x = gb[p][rr, pl.ds(16 * gq, 16)]
                            lo_f = lax.bitcast_convert_type(x << 16,
                                                            jnp.float32)
                            hi_f = lax.bitcast_convert_type(x & himask,
                                                            jnp.float32)
                            fb[p][2 * rr, pl.ds(16 * gq, 16)] = lo_f
                            fb[p][2 * rr + 1, pl.ds(16 * gq, 16)] = hi_f
                        return carry4

                    lax.fori_loop(0, BLK, cvt, 0)
                    start_scatter(p)

                @pl.when(b + 2 < nb)
                def _(b=b, p=p):
                    start_gather(b + 2, p)
            return carry3

        lax.fori_loop(0, nbp, pipe, 0)
        for p in (0, 1):
            @pl.when(p < nb)
            def _(p=p):
                wait_scatter(p)

        plsc.subcore_barrier()

        @pl.when(s < NS - 1)
        def _():
            pltpu.sync_copy(acc_sh.at[pl.ds(r0, INIT_ROWS)],
                            out_hbm.at[pl.ds(2 * lo + r0, INIT_ROWS)])

        @pl.when(s == NS - 1)
        def _():
            pltpu.sync_copy(
                acc_sh.at[pl.ds((NS - 1) * INIT_ROWS, INIT_ROWS_LAST)],
                out_hbm.at[pl.ds(2 * lo + (NS - 1) * INIT_ROWS,
                                 INIT_ROWS_LAST)])

        plsc.subcore_barrier()


def kernel(x, edge_index, W_conv, W_lin, b_lin):
    h_bf, base = _dense(x, W_conv[jnp.asarray(_PERM)], W_lin, b_lin)
    hbi = lax.bitcast_convert_type(h_bf.reshape(N_NODES, DI, 2), jnp.int32)
    src = edge_index[0].astype(jnp.int32)
    dst = edge_index[1].astype(jnp.int32)
    pad = E_PAD - E
    src_p = jnp.concatenate([src, jnp.zeros((pad,), jnp.int32)])
    dst_p = jnp.concatenate([dst, jnp.full((pad,), N_NODES, jnp.int32)])
    packed = src_p | (dst_p << 14)
    edges2d = packed.reshape(E_PAD // 128, 128)
    out = _sc_agg(hbi, base.reshape(2 * N_NODES, DI), edges2d)
    return out.reshape(N_NODES, D)


# speedup vs baseline: 1.9513x; 1.3674x over previous
import functools

import jax
import jax.numpy as jnp
from jax import lax
from jax.experimental import pallas as pl
from jax.experimental.pallas import tpu as pltpu
from jax.experimental.pallas import tpu_sc as plsc

N_NODES = 10000
D = 256
DH = 128
E = 160000
NS = 16
E_PAD = 163840
EDGES_PER_SUB = E_PAD // NS
SUB_CHUNK = 128
IDX_ROWS = 8
N_SUB = 2
CHUNK = SUB_CHUNK * IDX_ROWS
N_CHUNKS = EDGES_PER_SUB // CHUNK
ACC_ROWS = N_NODES + 48
OUT_ROWS = 640
OUT_ROWS_LAST = N_NODES - (NS - 1) * OUT_ROWS
ROW_BLK = 1000


def _dense_body(x_ref, wc_ref, wl_ref, b_ref, h_ref, base_ref):
    xb = x_ref[...]
    dn = (((1,), (1,)), ((), ()))
    h = lax.dot_general(xb, wc_ref[...], dn, preferred_element_type=jnp.float32)
    base = lax.dot_general(xb, wl_ref[...], dn, preferred_element_type=jnp.float32)
    base = base + b_ref[...]
    h_ref[0] = h[:, :DH]
    h_ref[1] = h[:, DH:]
    base_ref[0] = base[:, :DH]
    base_ref[1] = base[:, DH:]


def _dense(x, W_conv, W_lin, b_lin):
    return pl.pallas_call(
        _dense_body,
        grid=(N_NODES // ROW_BLK,),
        in_specs=[
            pl.BlockSpec((ROW_BLK, D), lambda i: (i, 0)),
            pl.BlockSpec((D, D), lambda i: (0, 0)),
            pl.BlockSpec((D, D), lambda i: (0, 0)),
            pl.BlockSpec((1, D), lambda i: (0, 0)),
        ],
        out_specs=[
            pl.BlockSpec((2, ROW_BLK, DH), lambda i: (0, i, 0)),
            pl.BlockSpec((2, ROW_BLK, DH), lambda i: (0, i, 0)),
        ],
        out_shape=[
            jax.ShapeDtypeStruct((2, N_NODES, DH), jnp.float32),
            jax.ShapeDtypeStruct((2, N_NODES, DH), jnp.float32),
        ],
    )(x, W_conv, W_lin, b_lin.reshape(1, D))


_sc_mesh = plsc.VectorSubcoreMesh(core_axis_name="c", subcore_axis_name="s")


@functools.partial(
    pl.kernel,
    out_type=jax.ShapeDtypeStruct((N_NODES, D), jnp.float32),
    mesh=_sc_mesh,
    scratch_types=[
        pltpu.VMEM((IDX_ROWS, SUB_CHUNK), jnp.int32),
        pltpu.VMEM((IDX_ROWS, SUB_CHUNK), jnp.int32),
        pltpu.VMEM((2 * SUB_CHUNK, DH), jnp.float32),
        pltpu.VMEM_SHARED((ACC_ROWS, DH), jnp.float32),
        pltpu.SemaphoreType.DMA,
        pltpu.SemaphoreType.DMA,
        pltpu.SemaphoreType.DMA,
        pltpu.SemaphoreType.DMA,
        pltpu.SemaphoreType.DMA,
        pltpu.SemaphoreType.DMA,
    ],
)
def _sc_agg(h_hbm, base_hbm, src_hbm, dst_hbm, out_hbm,
            src_v, dst_v, rows_v, acc_sh,
            gsem_a, gsem_b, gsem_c, gsem_d, ssem_a, ssem_b):
    c = lax.axis_index("c")
    s = lax.axis_index("s")
    r0 = s * OUT_ROWS

    @pl.when(s < NS - 1)
    def _():
        pltpu.sync_copy(base_hbm.at[c, pl.ds(r0, OUT_ROWS)],
                        acc_sh.at[pl.ds(r0, OUT_ROWS)])

    @pl.when(s == NS - 1)
    def _():
        pltpu.sync_copy(base_hbm.at[c, pl.ds((NS - 1) * OUT_ROWS, OUT_ROWS_LAST)],
                        acc_sh.at[pl.ds((NS - 1) * OUT_ROWS, OUT_ROWS_LAST)])

    plsc.subcore_barrier()

    idx_row0 = s * (EDGES_PER_SUB // SUB_CHUNK)
    gsems = (gsem_a, gsem_b, gsem_c, gsem_d)
    ssems = (ssem_a, ssem_b)
    HALF = SUB_CHUNK // 2

    def start_pair(j, src_ref):
        s0 = (2 * j) % 4
        g0 = pltpu.async_copy(
            h_hbm.at[c].at[src_ref.at[j, pl.ds(0, HALF)]],
            rows_v.at[pl.ds(s0 * HALF, HALF)], gsems[s0])
        g1 = pltpu.async_copy(
            h_hbm.at[c].at[src_ref.at[j, pl.ds(HALF, HALF)]],
            rows_v.at[pl.ds((s0 + 1) * HALF, HALF)], gsems[s0 + 1])
        return g0, g1

    def chunk_body(k, carry):
        r = idx_row0 + k * IDX_ROWS
        pltpu.sync_copy(src_hbm.at[pl.ds(r, IDX_ROWS)], src_v)
        pltpu.sync_copy(dst_hbm.at[pl.ds(r, IDX_ROWS)], dst_v)
        g = [None] * IDX_ROWS
        sc = [None] * IDX_ROWS
        g[0] = start_pair(0, src_v)
        for j in range(IDX_ROWS):
            if j + 1 < IDX_ROWS:
                if j >= 1:
                    sc[j - 1].wait()
                g[j + 1] = start_pair(j + 1, src_v)
            g[j][0].wait()
            g[j][1].wait()
            sc[j] = pltpu.async_copy(
                rows_v.at[pl.ds((j % 2) * SUB_CHUNK, SUB_CHUNK)],
                acc_sh.at[dst_v.at[j]], ssems[j % 2], add=True)
        sc[IDX_ROWS - 2].wait()
        sc[IDX_ROWS - 1].wait()
        return carry

    lax.fori_loop(0, N_CHUNKS, chunk_body, 0)
    plsc.subcore_barrier()

    @pl.when(s < NS - 1)
    def _():
        pltpu.sync_copy(acc_sh.at[pl.ds(r0, OUT_ROWS)],
                        out_hbm.at[pl.ds(r0, OUT_ROWS), pl.ds(c * DH, DH)])

    @pl.when(s == NS - 1)
    def _():
        pltpu.sync_copy(
            acc_sh.at[pl.ds((NS - 1) * OUT_ROWS, OUT_ROWS_LAST)],
            out_hbm.at[pl.ds((NS - 1) * OUT_ROWS, OUT_ROWS_LAST),
                       pl.ds(c * DH, DH)])


def kernel(x, edge_index, W_conv, W_lin, b_lin):
    h2, base2 = _dense(x, W_conv, W_lin, b_lin)
    src = edge_index[0].astype(jnp.int32)
    dst = edge_index[1].astype(jnp.int32)
    pad = E_PAD - E
    src_p = jnp.concatenate([src, jnp.zeros((pad,), jnp.int32)])
    dst_p = jnp.concatenate([dst, jnp.full((pad,), N_NODES, jnp.int32)])
    src2d = src_p.reshape(E_PAD // SUB_CHUNK, SUB_CHUNK)
    dst2d = dst_p.reshape(E_PAD // SUB_CHUNK, SUB_CHUNK)
    return _sc_agg(h2, base2, src2d, dst2d)


# speedup vs baseline: 2.0042x; 1.0271x over previous
import functools

import jax
import jax.numpy as jnp
from jax import lax
from jax.experimental import pallas as pl
from jax.experimental.pallas import tpu as pltpu
from jax.experimental.pallas import tpu_sc as plsc

N_NODES = 10000
D = 256
DH = 128
E = 160000
NS = 16
E_PAD = 163840
EDGES_PER_SUB = E_PAD // NS
SUB_CHUNK = 128
IDX_ROWS = 8
N_SUB = 2
CHUNK = SUB_CHUNK * IDX_ROWS
N_CHUNKS = EDGES_PER_SUB // CHUNK
ACC_ROWS = N_NODES + 48
OUT_ROWS = 640
OUT_ROWS_LAST = N_NODES - (NS - 1) * OUT_ROWS
ROW_BLK = 1000


def _dense_body(x_ref, wc_ref, wl_ref, b_ref, h_ref, base_ref):
    xb = x_ref[...]
    dn = (((1,), (1,)), ((), ()))
    h = lax.dot_general(xb, wc_ref[...], dn, preferred_element_type=jnp.float32)
    base = lax.dot_general(xb, wl_ref[...], dn, preferred_element_type=jnp.float32)
    base = base + b_ref[...]
    h_ref[0] = h[:, :DH]
    h_ref[1] = h[:, DH:]
    base_ref[0] = base[:, :DH]
    base_ref[1] = base[:, DH:]


def _dense(x, W_conv, W_lin, b_lin):
    return pl.pallas_call(
        _dense_body,
        grid=(N_NODES // ROW_BLK,),
        in_specs=[
            pl.BlockSpec((ROW_BLK, D), lambda i: (i, 0)),
            pl.BlockSpec((D, D), lambda i: (0, 0)),
            pl.BlockSpec((D, D), lambda i: (0, 0)),
            pl.BlockSpec((1, D), lambda i: (0, 0)),
        ],
        out_specs=[
            pl.BlockSpec((2, ROW_BLK, DH), lambda i: (0, i, 0)),
            pl.BlockSpec((2, ROW_BLK, DH), lambda i: (0, i, 0)),
        ],
        out_shape=[
            jax.ShapeDtypeStruct((2, N_NODES, DH), jnp.float32),
            jax.ShapeDtypeStruct((2, N_NODES, DH), jnp.float32),
        ],
    )(x, W_conv, W_lin, b_lin.reshape(1, D))


_sc_mesh = plsc.VectorSubcoreMesh(core_axis_name="c", subcore_axis_name="s")


@functools.partial(
    pl.kernel,
    out_type=jax.ShapeDtypeStruct((N_NODES, D), jnp.float32),
    mesh=_sc_mesh,
    scratch_types=[
        pltpu.VMEM((IDX_ROWS, SUB_CHUNK), jnp.int32),
        pltpu.VMEM((IDX_ROWS, SUB_CHUNK), jnp.int32),
        pltpu.VMEM((IDX_ROWS, SUB_CHUNK), jnp.int32),
        pltpu.VMEM((IDX_ROWS, SUB_CHUNK), jnp.int32),
        pltpu.VMEM((2 * SUB_CHUNK, DH), jnp.float32),
        pltpu.VMEM_SHARED((ACC_ROWS, DH), jnp.float32),
        pltpu.SemaphoreType.DMA,
        pltpu.SemaphoreType.DMA,
        pltpu.SemaphoreType.DMA,
        pltpu.SemaphoreType.DMA,
        pltpu.SemaphoreType.DMA,
        pltpu.SemaphoreType.DMA,
        pltpu.SemaphoreType.DMA,
        pltpu.SemaphoreType.DMA,
    ],
)
def _sc_agg(h_hbm, base_hbm, src_hbm, dst_hbm, out_hbm,
            src_va, dst_va, src_vb, dst_vb, rows_v, acc_sh,
            gsem_a, gsem_b, gsem_c, gsem_d, ssem_a, ssem_b,
            isem_a, isem_b):
    c = lax.axis_index("c")
    s = lax.axis_index("s")
    r0 = s * OUT_ROWS

    @pl.when(s < NS - 1)
    def _():
        pltpu.sync_copy(base_hbm.at[c, pl.ds(r0, OUT_ROWS)],
                        acc_sh.at[pl.ds(r0, OUT_ROWS)])

    @pl.when(s == NS - 1)
    def _():
        pltpu.sync_copy(base_hbm.at[c, pl.ds((NS - 1) * OUT_ROWS, OUT_ROWS_LAST)],
                        acc_sh.at[pl.ds((NS - 1) * OUT_ROWS, OUT_ROWS_LAST)])

    plsc.subcore_barrier()

    idx_row0 = s * (EDGES_PER_SUB // SUB_CHUNK)
    gsems = (gsem_a, gsem_b, gsem_c, gsem_d)
    ssems = (ssem_a, ssem_b)
    HALF = SUB_CHUNK // 2

    def start_pair(j, src_ref):
        s0 = (2 * j) % 4
        g0 = pltpu.async_copy(
            h_hbm.at[c].at[src_ref.at[j, pl.ds(0, HALF)]],
            rows_v.at[pl.ds(s0 * HALF, HALF)], gsems[s0])
        g1 = pltpu.async_copy(
            h_hbm.at[c].at[src_ref.at[j, pl.ds(HALF, HALF)]],
            rows_v.at[pl.ds((s0 + 1) * HALF, HALF)], gsems[s0 + 1])
        return g0, g1

    idx_bufs = ((src_va, dst_va), (src_vb, dst_vb))
    isems = (isem_a, isem_b)

    def start_idx_load(k, p):
        r = idx_row0 + k * IDX_ROWS
        pltpu.async_copy(src_hbm.at[pl.ds(r, IDX_ROWS)], idx_bufs[p][0],
                         isems[p])
        pltpu.async_copy(dst_hbm.at[pl.ds(r, IDX_ROWS)], idx_bufs[p][1],
                         isems[p])

    def wait_idx_load(k, p):
        r = idx_row0 + k * IDX_ROWS
        pltpu.make_async_copy(src_hbm.at[pl.ds(r, IDX_ROWS)], idx_bufs[p][0],
                              isems[p]).wait()
        pltpu.make_async_copy(dst_hbm.at[pl.ds(r, IDX_ROWS)], idx_bufs[p][1],
                              isems[p]).wait()

    def process_chunk(k, p):
        src_v, dst_v = idx_bufs[p]
        g = [None] * IDX_ROWS
        sc = [None] * IDX_ROWS
        g[0] = start_pair(0, src_v)
        for j in range(IDX_ROWS):
            if j + 1 < IDX_ROWS:
                if j >= 1:
                    sc[j - 1].wait()
                g[j + 1] = start_pair(j + 1, src_v)
            g[j][0].wait()
            g[j][1].wait()
            sc[j] = pltpu.async_copy(
                rows_v.at[pl.ds((j % 2) * SUB_CHUNK, SUB_CHUNK)],
                acc_sh.at[dst_v.at[j]], ssems[j % 2], add=True)
        sc[IDX_ROWS - 2].wait()
        sc[IDX_ROWS - 1].wait()

    start_idx_load(0, 0)

    def chunk_pair(jj, carry):
        k0 = 2 * jj
        wait_idx_load(k0, 0)
        start_idx_load(k0 + 1, 1)
        process_chunk(k0, 0)
        wait_idx_load(k0 + 1, 1)

        @pl.when(jj < N_CHUNKS // 2 - 1)
        def _():
            start_idx_load(k0 + 2, 0)

        process_chunk(k0 + 1, 1)
        return carry

    lax.fori_loop(0, N_CHUNKS // 2, chunk_pair, 0)
    plsc.subcore_barrier()

    @pl.when(s < NS - 1)
    def _():
        pltpu.sync_copy(acc_sh.at[pl.ds(r0, OUT_ROWS)],
                        out_hbm.at[pl.ds(r0, OUT_ROWS), pl.ds(c * DH, DH)])

    @pl.when(s == NS - 1)
    def _():
        pltpu.sync_copy(
            acc_sh.at[pl.ds((NS - 1) * OUT_ROWS, OUT_ROWS_LAST)],
            out_hbm.at[pl.ds((NS - 1) * OUT_ROWS, OUT_ROWS_LAST),
                       pl.ds(c * DH, DH)])


def kernel(x, edge_index, W_conv, W_lin, b_lin):
    h2, base2 = _dense(x, W_conv, W_lin, b_lin)
    src = edge_index[0].astype(jnp.int32)
    dst = edge_index[1].astype(jnp.int32)
    pad = E_PAD - E
    src_p = jnp.concatenate([src, jnp.zeros((pad,), jnp.int32)])
    dst_p = jnp.concatenate([dst, jnp.full((pad,), N_NODES, jnp.int32)])
    src2d = src_p.reshape(E_PAD // SUB_CHUNK, SUB_CHUNK)
    dst2d = dst_p.reshape(E_PAD // SUB_CHUNK, SUB_CHUNK)
    return _sc_agg(h2, base2, src2d, dst2d)
